# baseline (speedup 1.0000x reference)
import functools

import jax
import jax.numpy as jnp
from jax import lax
from jax.experimental import pallas as pl
from jax.experimental.pallas import tpu as pltpu
from jax.experimental.pallas import tpu_sc as plsc

NC = 2
NS = 16
NW = NC * NS

V = 1000000
D = 32
BATCH = 4096
HIST = 200

NTC_FULL = V // 128
TAIL = V - NTC_FULL * 128
RPASS = 123

NBLK = BATCH // 128
NITEM = HIST * NBLK
IPW = NITEM // NW


def _iota16():
  return lax.broadcasted_iota(jnp.int32, (16,), 0)


def _relayout():
  mesh = plsc.VectorSubcoreMesh(core_axis_name="c", subcore_axis_name="s")

  scratch = (
      [pltpu.VMEM((4, 8, 128), jnp.float32) for _ in range(2)]
      + [pltpu.VMEM((4096,), jnp.float32) for _ in range(2)]
      + [pltpu.SemaphoreType.DMA for _ in range(4)]
  )

  @functools.partial(
      pl.kernel,
      mesh=mesh,
      out_type=jax.ShapeDtypeStruct((V * D,), jnp.float32),
      scratch_types=scratch,
      compiler_params=pltpu.CompilerParams(use_tc_tiling_on_sc=True, needs_layout_passes=False),
  )
  def relayout(wt_hbm, wtail_hbm, out_hbm, *refs):
    slab_v = refs[0:2]
    lin_v = refs[2:4]
    in_sem = refs[4:6]
    out_sem = refs[6:8]

    wid = lax.axis_index("s") * NC + lax.axis_index("c")
    iota32 = _iota16() * 32

    def slab_start(c, s):
      for tr in range(4):
        pltpu.async_copy(wt_hbm.at[pl.ds(tr * 8, 8), pl.ds(c * 128, 128)],
                         slab_v[s].at[tr], in_sem[s])

    def slab_wait(s):
      for tr in range(4):
        pltpu.make_async_copy(wt_hbm.at[pl.ds(0, 8), pl.ds(0, 128)],
                              slab_v[s].at[tr], in_sem[s]).wait()

    def transpose(s, width):
      for f in range(D):
        tr, r = f // 8, f % 8
        for k8 in range(width // 16):
          x = slab_v[s][tr, r, pl.ds(k8 * 16, 16)]
          plsc.store_scatter(lin_v[s], [iota32 + (k8 * 512 + f)], x)

    def out_start(c, s):
      pltpu.async_copy(lin_v[s], out_hbm.at[pl.ds(c * 4096, 4096)],
                       out_sem[s])

    def out_wait(s):
      pltpu.make_async_copy(lin_v[s], out_hbm.at[pl.ds(0, 4096)],
                            out_sem[s]).wait()

    for s in range(2):
      pl.when(s * NW + wid < NTC_FULL)(
          functools.partial(slab_start, s * NW + wid, s))

    def ring(r, carry):
      for b in range(2):
        k = r * 2 + b
        c = k * NW + wid

        def item(c=c, b=b, k=k, r=r):
          slab_wait(b)
          pl.when(r >= 1)(lambda: out_wait(b))
          transpose(b, 128)
          out_start(c, b)
          nxt = c + 2 * NW
          pl.when(nxt < NTC_FULL)(
              functools.partial(slab_start, nxt, b))

        pl.when(c < NTC_FULL)(item)
      return carry

    lax.fori_loop(0, RPASS, ring, 0)
    for s in range(2):
      out_wait(s)

    def tail():
      pltpu.sync_copy(wtail_hbm, lin_v[0].at[pl.ds(0, TAIL * D)])
      pltpu.sync_copy(lin_v[0].at[pl.ds(0, TAIL * D)],
                      out_hbm.at[pl.ds(NTC_FULL * 4096, TAIL * D)])
    pl.when(wid == 4)(tail)

  return relayout


def _gather():
  mesh = plsc.VectorSubcoreMesh(core_axis_name="c", subcore_axis_name="s")

  scratch = (
      [pltpu.VMEM((128,), jnp.int32) for _ in range(2)]
      + [pltpu.VMEM((128, D), jnp.float32) for _ in range(2)]
      + [pltpu.VMEM((4096,), jnp.float32) for _ in range(2)]
      + [pltpu.SemaphoreType.DMA for _ in range(6)]
  )

  @functools.partial(
      pl.kernel,
      mesh=mesh,
      out_type=jax.ShapeDtypeStruct((HIST * D * BATCH,), jnp.float32),
      scratch_types=scratch,
      compiler_params=pltpu.CompilerParams(use_tc_tiling_on_sc=False, needs_layout_passes=False),
  )
  def gather(table_hbm, xt_hbm, out_hbm, *refs):
    idx_v = refs[0:2]
    rows_v = refs[2:4]
    tile_v = refs[4:6]
    i_sem = refs[6:8]
    g_sem = refs[8:10]
    o_sem = refs[10:12]

    wid = lax.axis_index("s") * NC + lax.axis_index("c")
    base = wid * IPW
    f16 = _iota16()
    foff = [((f16 + 16 * half) >> 3) * 1024 + ((f16 + 16 * half) & 7) * 128
            for half in range(2)]

    def idx_start(t, s):
      pltpu.async_copy(xt_hbm.at[t // NBLK, pl.ds((t % NBLK) * 128, 128)],
                       idx_v[s], i_sem[s])

    def idx_wait(s):
      pltpu.make_async_copy(xt_hbm.at[0, pl.ds(0, 128)], idx_v[s],
                            i_sem[s]).wait()

    def gath_start(s):
      pltpu.async_copy(table_hbm.at[idx_v[s]], rows_v[s], g_sem[s])

    def gath_wait(s):
      pltpu.make_async_copy(table_hbm.at[idx_v[s]], rows_v[s],
                            g_sem[s]).wait()

    def transpose(s):
      for j in range(128):
        for half in range(2):
          x = rows_v[s][j, pl.ds(half * 16, 16)]
          plsc.store_scatter(tile_v[s], [foff[half] + j], x)

    def out_start(t, s):
      h = t // NBLK
      bb = t % NBLK
      for fh in range(4):
        off = ((h * 4 + fh) * NBLK + bb) * 1024
        pltpu.async_copy(tile_v[s].at[pl.ds(fh * 1024, 1024)],
                         out_hbm.at[pl.ds(off, 1024)], o_sem[s])

    def out_wait(s):
      for fh in range(4):
        pltpu.make_async_copy(tile_v[s].at[pl.ds(0, 1024)],
                              out_hbm.at[pl.ds(0, 1024)], o_sem[s]).wait()

    for s in range(2):
      idx_start(base + s, s)

    def ring(r, carry):
      for b in range(2):
        k = r * 2 + b
        t = base + k
        idx_wait(b)
        gath_start(b)
        bp = 1 - b

        def retire(k=k, t=t, b=b, bp=bp):
          gath_wait(bp)
          pl.when(k + 1 < IPW)(functools.partial(idx_start, t + 1, bp))
          transpose(bp)
          out_start(t - 1, bp)

        def out_drain(bp=bp):
          out_wait(bp)

        if b == 0:
          pl.when(r >= 2)(out_drain)
          pl.when(r >= 1)(retire)
        else:
          pl.when(r >= 1)(out_drain)
          retire()
      return carry

    lax.fori_loop(0, IPW // 2, ring, 0)

    s_last = (IPW - 1) % 2
    gath_wait(s_last)
    out_wait(s_last)
    transpose(s_last)
    out_start(IPW - 1 + base, s_last)
    for s in range(2):
      out_wait(s)

  return gather


_relayout_k = _relayout()
_gather_k = _gather()


@jax.jit
def kernel(X, W):
  w1d = _relayout_k(W.T, W[NTC_FULL * 128:].reshape(-1))
  out1 = _gather_k(w1d.reshape(V, D), X.T)
  o5 = out1.reshape(HIST, 4, NBLK, 8, 128)
  return o5.transpose(2, 4, 0, 1, 3).reshape(BATCH, HIST, D)


# speedup vs baseline: 3.6815x; 3.6815x over previous
import functools

import jax
import jax.numpy as jnp
from jax import lax
from jax.experimental import pallas as pl
from jax.experimental.pallas import tpu as pltpu
from jax.experimental.pallas import tpu_sc as plsc

NC = 2
NS = 16
NW = NC * NS

V = 1000000
D = 32
BATCH = 4096
HIST = 200

NTC_FULL = V // 128
TAIL = V - NTC_FULL * 128
RPASS = 123

NBLK = BATCH // 128
NITEM = HIST * NBLK
IPW = NITEM // NW


def _iota16():
  return lax.broadcasted_iota(jnp.int32, (16,), 0)


def _relayout():
  mesh = plsc.VectorSubcoreMesh(core_axis_name="c", subcore_axis_name="s")

  scratch = (
      [pltpu.VMEM((4, 8, 128), jnp.float32) for _ in range(2)]
      + [pltpu.VMEM((4096,), jnp.float32) for _ in range(2)]
      + [pltpu.SemaphoreType.DMA for _ in range(4)]
  )

  @functools.partial(
      pl.kernel,
      mesh=mesh,
      out_type=jax.ShapeDtypeStruct((V * D,), jnp.float32),
      scratch_types=scratch,
      compiler_params=pltpu.CompilerParams(use_tc_tiling_on_sc=True, needs_layout_passes=False),
  )
  def relayout(wt_hbm, wtail_hbm, out_hbm, *refs):
    slab_v = refs[0:2]
    lin_v = refs[2:4]
    in_sem = refs[4:6]
    out_sem = refs[6:8]

    wid = lax.axis_index("s") * NC + lax.axis_index("c")
    iota32 = _iota16() * 32

    def slab_start(c, s):
      for tr in range(4):
        pltpu.async_copy(wt_hbm.at[pl.ds(tr * 8, 8), pl.ds(c * 128, 128)],
                         slab_v[s].at[tr], in_sem[s])

    def slab_wait(s):
      for tr in range(4):
        pltpu.make_async_copy(wt_hbm.at[pl.ds(0, 8), pl.ds(0, 128)],
                              slab_v[s].at[tr], in_sem[s]).wait()

    def transpose(s):
      @functools.partial(plsc.parallel_loop, 0, 256, unroll=8)
      def body(q):
        f = q >> 3
        k8 = q & 7
        x = slab_v[s][f >> 3, f & 7, pl.ds(k8 * 16, 16)]
        plsc.store_scatter(lin_v[s], [iota32 + (k8 * 512 + f)], x)

    def out_start(c, s):
      pltpu.async_copy(lin_v[s], out_hbm.at[pl.ds(c * 4096, 4096)],
                       out_sem[s])

    def out_wait(s):
      pltpu.make_async_copy(lin_v[s], out_hbm.at[pl.ds(0, 4096)],
                            out_sem[s]).wait()

    for s in range(2):
      pl.when(s * NW + wid < NTC_FULL)(
          functools.partial(slab_start, s * NW + wid, s))

    def ring(r, carry):
      for b in range(2):
        k = r * 2 + b
        c = k * NW + wid

        def item(c=c, b=b, k=k, r=r):
          slab_wait(b)
          pl.when(r >= 1)(lambda: out_wait(b))
          transpose(b)
          out_start(c, b)
          nxt = c + 2 * NW
          pl.when(nxt < NTC_FULL)(
              functools.partial(slab_start, nxt, b))

        pl.when(c < NTC_FULL)(item)
      return carry

    lax.fori_loop(0, RPASS, ring, 0)
    for s in range(2):
      out_wait(s)

    def tail():
      pltpu.sync_copy(wtail_hbm, lin_v[0].at[pl.ds(0, TAIL * D)])
      pltpu.sync_copy(lin_v[0].at[pl.ds(0, TAIL * D)],
                      out_hbm.at[pl.ds(NTC_FULL * 4096, TAIL * D)])
    pl.when(wid == 4)(tail)

  return relayout


def _gather():
  mesh = plsc.VectorSubcoreMesh(core_axis_name="c", subcore_axis_name="s")

  scratch = (
      [pltpu.VMEM((128,), jnp.int32) for _ in range(2)]
      + [pltpu.VMEM((128, D), jnp.float32) for _ in range(2)]
      + [pltpu.VMEM((4096,), jnp.float32) for _ in range(2)]
      + [pltpu.SemaphoreType.DMA for _ in range(6)]
  )

  @functools.partial(
      pl.kernel,
      mesh=mesh,
      out_type=jax.ShapeDtypeStruct((HIST * D * BATCH,), jnp.float32),
      scratch_types=scratch,
      compiler_params=pltpu.CompilerParams(use_tc_tiling_on_sc=False, needs_layout_passes=False),
  )
  def gather(table_hbm, xt_hbm, out_hbm, *refs):
    idx_v = refs[0:2]
    rows_v = refs[2:4]
    tile_v = refs[4:6]
    i_sem = refs[6:8]
    g_sem = refs[8:10]
    o_sem = refs[10:12]

    wid = lax.axis_index("s") * NC + lax.axis_index("c")
    base = wid * IPW
    f16 = _iota16()
    foff = [((f16 + 16 * half) >> 3) * 1024 + ((f16 + 16 * half) & 7) * 128
            for half in range(2)]

    def idx_start(t, s):
      pltpu.async_copy(xt_hbm.at[t // NBLK, pl.ds((t % NBLK) * 128, 128)],
                       idx_v[s], i_sem[s])

    def idx_wait(s):
      pltpu.make_async_copy(xt_hbm.at[0, pl.ds(0, 128)], idx_v[s],
                            i_sem[s]).wait()

    def gath_start(s):
      pltpu.async_copy(table_hbm.at[idx_v[s]], rows_v[s], g_sem[s])

    def gath_wait(s):
      pltpu.make_async_copy(table_hbm.at[idx_v[s]], rows_v[s],
                            g_sem[s]).wait()

    def transpose(s):
      @functools.partial(plsc.parallel_loop, 0, 128, unroll=8)
      def body(j):
        x0 = rows_v[s][j, pl.ds(0, 16)]
        plsc.store_scatter(tile_v[s], [foff[0] + j], x0)
        x1 = rows_v[s][j, pl.ds(16, 16)]
        plsc.store_scatter(tile_v[s], [foff[1] + j], x1)

    def out_start(t, s):
      h = t // NBLK
      bb = t % NBLK
      for fh in range(4):
        off = ((h * 4 + fh) * NBLK + bb) * 1024
        pltpu.async_copy(tile_v[s].at[pl.ds(fh * 1024, 1024)],
                         out_hbm.at[pl.ds(off, 1024)], o_sem[s])

    def out_wait(s):
      for fh in range(4):
        pltpu.make_async_copy(tile_v[s].at[pl.ds(0, 1024)],
                              out_hbm.at[pl.ds(0, 1024)], o_sem[s]).wait()

    for s in range(2):
      idx_start(base + s, s)

    def ring(r, carry):
      for b in range(2):
        k = r * 2 + b
        t = base + k
        idx_wait(b)
        gath_start(b)
        bp = 1 - b

        def retire(k=k, t=t, b=b, bp=bp):
          gath_wait(bp)
          pl.when(k + 1 < IPW)(functools.partial(idx_start, t + 1, bp))
          transpose(bp)
          out_start(t - 1, bp)

        def out_drain(bp=bp):
          out_wait(bp)

        if b == 0:
          pl.when(r >= 2)(out_drain)
          pl.when(r >= 1)(retire)
        else:
          pl.when(r >= 1)(out_drain)
          retire()
      return carry

    lax.fori_loop(0, IPW // 2, ring, 0)

    s_last = (IPW - 1) % 2
    gath_wait(s_last)
    out_wait(s_last)
    transpose(s_last)
    out_start(IPW - 1 + base, s_last)
    for s in range(2):
      out_wait(s)

  return gather


_relayout_k = _relayout()
_gather_k = _gather()


@jax.jit
def kernel(X, W):
  w1d = _relayout_k(W.T, W[NTC_FULL * 128:].reshape(-1))
  out1 = _gather_k(w1d.reshape(V, D), X.T)
  o5 = out1.reshape(HIST, 4, NBLK, 8, 128)
  return o5.transpose(2, 4, 0, 1, 3).reshape(BATCH, HIST, D)
